# Initial kernel scaffold; baseline (speedup 1.0000x reference)
import jax
import jax.numpy as jnp
from jax.experimental import pallas as pl


def kernel(x, W_proj_packed, W_Out_packed):
    raise NotImplementedError("write your pallas kernel here")


# speedup vs baseline: 2.9926x; 2.9926x over previous
import functools

import jax
import jax.numpy as jnp
from jax import lax
from jax.experimental import pallas as pl
from jax.experimental.pallas import tpu as pltpu

_NUM_HEAD = 16
_HEAD_DIM = 64
_SEG = 1024


def _mhsa_kernel(x_ref, wp_ref, wo_ref, o_ref, proj_ref):
    xb = x_ref[...].astype(jnp.bfloat16)
    proj_ref[...] = jnp.dot(
        xb, wp_ref[...], preferred_element_type=jnp.float32
    ).astype(jnp.bfloat16)

    pvs = []
    for h in range(_NUM_HEAD):
        q_sl = slice(h * _HEAD_DIM, (h + 1) * _HEAD_DIM)
        k_sl = slice(_SEG + h * _HEAD_DIM, _SEG + (h + 1) * _HEAD_DIM)
        v_sl = slice(2 * _SEG + h * _HEAD_DIM, 2 * _SEG + (h + 1) * _HEAD_DIM)
        s = lax.dot_general(
            proj_ref[:, q_sl], proj_ref[:, k_sl],
            (((1,), (1,)), ((), ())),
            preferred_element_type=jnp.float32)
        m = jnp.max(s, axis=-1, keepdims=True)
        p = jnp.exp(s - m)
        v_ext = jnp.concatenate(
            [proj_ref[:, v_sl].astype(jnp.float32),
             jnp.ones((p.shape[1], _HEAD_DIM), jnp.float32)], axis=1)
        pv = jnp.dot(p, v_ext, preferred_element_type=jnp.float32)
        l = pv[:, _HEAD_DIM:_HEAD_DIM + 1]
        pvs.append((pv[:, :_HEAD_DIM] * (1.0 / l)).astype(jnp.bfloat16))

    acc = jnp.concatenate(pvs, axis=1)
    o_ref[...] = jnp.dot(acc, wo_ref[...],
                         preferred_element_type=jnp.float32)


def kernel(x, W_proj_packed, W_Out_packed):
    bsz, slen, embed_dim = x.shape
    seg = _SEG
    out = pl.pallas_call(
        _mhsa_kernel,
        out_shape=jax.ShapeDtypeStruct((bsz, slen, embed_dim), jnp.float32),
        grid=(bsz,),
        in_specs=[
            pl.BlockSpec((None, slen, embed_dim), lambda b: (b, 0, 0)),
            pl.BlockSpec((embed_dim, 3 * seg), lambda b: (0, 0)),
            pl.BlockSpec((seg, embed_dim), lambda b: (0, 0)),
        ],
        out_specs=pl.BlockSpec((None, slen, embed_dim), lambda b: (b, 0, 0)),
        scratch_shapes=[
            pltpu.VMEM((slen, 3 * seg), jnp.bfloat16),
        ],
        compiler_params=pltpu.CompilerParams(
            dimension_semantics=("parallel",),
            vmem_limit_bytes=64 * 1024 * 1024),
    )(x, W_proj_packed, W_Out_packed)
    return out


# speedup vs baseline: 3.1405x; 1.0494x over previous
import functools

import jax
import jax.numpy as jnp
from jax import lax
from jax.experimental import pallas as pl
from jax.experimental.pallas import tpu as pltpu

_NUM_HEAD = 16
_HEAD_DIM = 64
_SEG = 1024
_BPP = 2


def _mhsa_kernel(x_ref, wp_ref, wo_ref, o_ref, proj_ref):
    slen = x_ref.shape[1]
    xb = x_ref[...].astype(jnp.bfloat16).reshape(_BPP * slen, -1)
    proj_ref[...] = jnp.dot(
        xb, wp_ref[...], preferred_element_type=jnp.float32
    ).astype(jnp.bfloat16)

    pvs = [[] for _ in range(_BPP)]
    for h in range(_NUM_HEAD):
        q_sl = slice(h * _HEAD_DIM, (h + 1) * _HEAD_DIM)
        k_sl = slice(_SEG + h * _HEAD_DIM, _SEG + (h + 1) * _HEAD_DIM)
        v_sl = slice(2 * _SEG + h * _HEAD_DIM, 2 * _SEG + (h + 1) * _HEAD_DIM)
        for b in range(_BPP):
            r_sl = slice(b * slen, (b + 1) * slen)
            s = lax.dot_general(
                proj_ref[r_sl, q_sl], proj_ref[r_sl, k_sl],
                (((1,), (1,)), ((), ())),
                preferred_element_type=jnp.float32)
            m = jnp.max(s, axis=-1, keepdims=True)
            p = jnp.exp(s - m)
            v_ext = jnp.concatenate(
                [proj_ref[r_sl, v_sl].astype(jnp.float32),
                 jnp.ones((slen, _HEAD_DIM), jnp.float32)], axis=1)
            pv = jnp.dot(p, v_ext, preferred_element_type=jnp.float32)
            l = pv[:, _HEAD_DIM:_HEAD_DIM + 1]
            pvs[b].append((pv[:, :_HEAD_DIM] * (1.0 / l)).astype(jnp.bfloat16))

    acc = jnp.concatenate(
        [jnp.concatenate(pvs[b], axis=1) for b in range(_BPP)], axis=0)
    out = jnp.dot(acc, wo_ref[...], preferred_element_type=jnp.float32)
    o_ref[...] = out.reshape(_BPP, slen, -1)


def kernel(x, W_proj_packed, W_Out_packed):
    bsz, slen, embed_dim = x.shape
    seg = _SEG
    out = pl.pallas_call(
        _mhsa_kernel,
        out_shape=jax.ShapeDtypeStruct((bsz, slen, embed_dim), jnp.float32),
        grid=(bsz // _BPP,),
        in_specs=[
            pl.BlockSpec((_BPP, slen, embed_dim), lambda b: (b, 0, 0)),
            pl.BlockSpec((embed_dim, 3 * seg), lambda b: (0, 0)),
            pl.BlockSpec((seg, embed_dim), lambda b: (0, 0)),
        ],
        out_specs=pl.BlockSpec((_BPP, slen, embed_dim), lambda b: (b, 0, 0)),
        scratch_shapes=[
            pltpu.VMEM((_BPP * slen, 3 * seg), jnp.bfloat16),
        ],
        compiler_params=pltpu.CompilerParams(
            dimension_semantics=("parallel",),
            vmem_limit_bytes=64 * 1024 * 1024),
    )(x, W_proj_packed, W_Out_packed)
    return out
